# baseline (speedup 1.0000x reference)
import functools

import jax
import jax.numpy as jnp
from jax import lax
from jax.experimental import pallas as pl
from jax.experimental.pallas import tpu as pltpu
from jax.experimental.pallas import tpu_sc as plsc

F = 64
L = 16
TC = 128
CHUNK = 256


def _stage_scalars(idx_v, dst_s, n, transform):
    lanes = lax.iota(jnp.int32, L)

    def stage(g, carry):
        vec = transform(idx_v[pl.ds(g * L, L)])
        zero = jnp.zeros((L,), jnp.int32)
        for j in range(L):
            dst_s[g * L + j] = jnp.sum(jnp.where(lanes == j, vec, zero))
        return carry

    lax.fori_loop(0, n // L, stage, 0)


@jax.jit
def kernel(user, item, user_factors, item_factors):
    B = user.shape[0]
    info = plsc.get_sparse_core_info()
    NC = info.num_cores
    NW = NC * info.num_subcores
    b_per_w = B // NW

    if_t = item_factors.T
    uf_packed = user_factors.reshape(user_factors.shape[0] // 2, 2 * F)
    mesh = plsc.VectorSubcoreMesh(core_axis_name="c", subcore_axis_name="s")

    G = 4

    @functools.partial(
        pl.kernel,
        mesh=mesh,
        out_type=jax.ShapeDtypeStruct((B // 2, 2 * F), jnp.float32),
        compiler_params=pltpu.CompilerParams(needs_layout_passes=False),
        scratch_types=[
            pltpu.VMEM((b_per_w,), jnp.int32),
            pltpu.SMEM((b_per_w,), jnp.int32),
            pltpu.SMEM((b_per_w,), jnp.int32),
            pltpu.VMEM((G, F, TC), jnp.float32),
            pltpu.VMEM((G, F, TC), jnp.float32),
            pltpu.VMEM((b_per_w // 2, 2 * F), jnp.float32),
            pltpu.SemaphoreType.DMA,
            pltpu.SemaphoreType.DMA,
        ],
    )
    def gather_cols(item_hbm, ift_hbm, rows_hbm,
                    iidx_v, blk_s, sub_s, buf_a, buf_b, rows_v, sem_a, sem_b):
        wid = lax.axis_index("s") * NC + lax.axis_index("c")
        base = wid * b_per_w
        pltpu.sync_copy(item_hbm.at[pl.ds(base, b_per_w)], iidx_v)
        _stage_scalars(iidx_v, blk_s, b_per_w,
                       lambda v: lax.shift_right_logical(v, 7))
        _stage_scalars(iidx_v, sub_s, b_per_w,
                       lambda v: jnp.bitwise_and(v, jnp.full((L,), TC - 1,
                                                             jnp.int32)))

        def fetch_group(e0, buf, sem):
            for j in range(G):
                off = pl.multiple_of(blk_s[e0 + j] * TC, TC)
                pltpu.async_copy(ift_hbm.at[:, pl.ds(off, TC)],
                                 buf.at[j], sem)

        def drain_group(buf, sem):
            for j in range(G):
                pltpu.make_async_copy(ift_hbm.at[:, pl.ds(0, TC)],
                                      buf.at[j], sem).wait()

        def extract_group(e0, buf):
            for j in range(G):
                e = e0 + j
                sub = jnp.full((L,), sub_s[e], jnp.int32)
                jvec = jnp.full((L,), j, jnp.int32)
                erow = jnp.full((L,), lax.shift_right_logical(e, 1),
                                jnp.int32)
                cbase = jnp.bitwise_and(e, 1) * F
                for k in range(F // L):
                    fvec = k * L + lax.iota(jnp.int32, L)
                    vals = plsc.load_gather(buf, [jvec, fvec, sub])
                    plsc.store_scatter(rows_v, [erow, cbase + fvec], vals)

        n_pairs = b_per_w // (2 * G)
        fetch_group(0, buf_a, sem_a)

        def body(p, carry):
            e = p * 2 * G
            fetch_group(e + G, buf_b, sem_b)
            drain_group(buf_a, sem_a)
            extract_group(e, buf_a)

            @pl.when(p + 1 < n_pairs)
            def _():
                fetch_group(e + 2 * G, buf_a, sem_a)

            drain_group(buf_b, sem_b)
            extract_group(e + G, buf_b)
            return carry

        lax.fori_loop(0, n_pairs, body, 0)
        pltpu.sync_copy(rows_v,
                        rows_hbm.at[pl.ds(pl.multiple_of(base // 2, 8), b_per_w // 2), :])

    @functools.partial(
        pl.kernel,
        mesh=mesh,
        out_type=jax.ShapeDtypeStruct((B,), jnp.float32),
        compiler_params=pltpu.CompilerParams(needs_layout_passes=False),
        scratch_types=[
            pltpu.VMEM((b_per_w,), jnp.int32),
            pltpu.SMEM((b_per_w,), jnp.int32),
            pltpu.VMEM((CHUNK, 2 * F), jnp.float32),
            pltpu.VMEM((b_per_w // 2, 2 * F), jnp.float32),
            pltpu.VMEM((b_per_w,), jnp.float32),
            pltpu.SemaphoreType.DMA,
            pltpu.SemaphoreType.DMA,
        ],
    )
    def dot_rows(user_hbm, uf_hbm, vrows_hbm, out_hbm,
                 uidx_v, uidx_s, urows, vrows_v, out_v, sem_u, sem_v):
        wid = lax.axis_index("s") * NC + lax.axis_index("c")
        base = wid * b_per_w
        cp_v = pltpu.async_copy(
            vrows_hbm.at[pl.ds(pl.multiple_of(base // 2, 8), b_per_w // 2), :], vrows_v, sem_v)
        pltpu.sync_copy(user_hbm.at[pl.ds(base, b_per_w)], uidx_v)
        _stage_scalars(uidx_v, uidx_s, b_per_w,
                       lambda v: lax.shift_right_logical(v, 1))
        cp_v.wait()

        for c in range(b_per_w // CHUNK):

            def issue(i, carry):
                ur = uidx_s[c * CHUNK + i]
                pltpu.async_copy(uf_hbm.at[pl.ds(ur, 1), :],
                                 urows.at[pl.ds(i, 1), :], sem_u)
                return carry

            lax.fori_loop(0, CHUNK, issue, 0)
            pltpu.make_async_copy(uf_hbm.at[pl.ds(0, CHUNK), :], urows,
                                  sem_u).wait()

            for g in range(CHUNK // L):
                rows = g * L + lax.iota(jnp.int32, L)
                il = c * CHUNK + g * L + lax.iota(jnp.int32, L)
                uvec = uidx_v[pl.ds(c * CHUNK + g * L, L)]
                ubase = jnp.bitwise_and(uvec, jnp.full((L,), 1, jnp.int32)) * F
                vrow = lax.shift_right_logical(il, 1)
                vbase = jnp.bitwise_and(il, jnp.full((L,), 1, jnp.int32)) * F
                accs = [jnp.zeros((L,), jnp.float32) for _ in range(4)]
                for f in range(F):
                    colf = jnp.full((L,), f, jnp.int32)
                    uc = plsc.load_gather(urows, [rows, ubase + colf])
                    vc = plsc.load_gather(vrows_v, [vrow, vbase + colf])
                    accs[f % 4] = accs[f % 4] + uc * vc
                acc = (accs[0] + accs[1]) + (accs[2] + accs[3])
                out_v[pl.ds(c * CHUNK + g * L, L)] = acc

        pltpu.sync_copy(out_v, out_hbm.at[pl.ds(base, b_per_w)])

    item_rows = gather_cols(item, if_t)
    return dot_rows(user, uf_packed, item_rows)


# speedup vs baseline: 1.5161x; 1.5161x over previous
import functools

import jax
import jax.numpy as jnp
from jax import lax
from jax.experimental import pallas as pl
from jax.experimental.pallas import tpu as pltpu
from jax.experimental.pallas import tpu_sc as plsc

F = 64
L = 16
TC = 128
CHUNK = 256


def _stage_scalars(idx_v, dst_s, n, transform):
    lanes = lax.iota(jnp.int32, L)

    def stage(g, carry):
        vec = transform(idx_v[pl.ds(g * L, L)])
        zero = jnp.zeros((L,), jnp.int32)
        for j in range(L):
            dst_s[g * L + j] = jnp.sum(jnp.where(lanes == j, vec, zero))
        return carry

    lax.fori_loop(0, n // L, stage, 0)


@jax.jit
def kernel(user, item, user_factors, item_factors):
    B = user.shape[0]
    info = plsc.get_sparse_core_info()
    NC = info.num_cores
    NW = NC * info.num_subcores
    b_per_w = B // NW

    if_t = item_factors.T
    mesh = plsc.VectorSubcoreMesh(core_axis_name="c", subcore_axis_name="s")

    G = 4

    @functools.partial(
        pl.kernel,
        mesh=mesh,
        out_type=jax.ShapeDtypeStruct((B // 2, 2 * F), jnp.float32),
        compiler_params=pltpu.CompilerParams(needs_layout_passes=False),
        scratch_types=[
            pltpu.VMEM((b_per_w,), jnp.int32),
            pltpu.SMEM((b_per_w,), jnp.int32),
            pltpu.SMEM((b_per_w,), jnp.int32),
            pltpu.VMEM((G, F, TC), jnp.float32),
            pltpu.VMEM((G, F, TC), jnp.float32),
            pltpu.VMEM((b_per_w // 2, 2 * F), jnp.float32),
            pltpu.SemaphoreType.DMA,
            pltpu.SemaphoreType.DMA,
        ],
    )
    def gather_cols(item_hbm, ift_hbm, rows_hbm,
                    iidx_v, blk_s, sub_s, buf_a, buf_b, rows_v, sem_a, sem_b):
        wid = lax.axis_index("s") * NC + lax.axis_index("c")
        base = wid * b_per_w
        pltpu.sync_copy(item_hbm.at[pl.ds(base, b_per_w)], iidx_v)
        _stage_scalars(iidx_v, blk_s, b_per_w,
                       lambda v: lax.shift_right_logical(v, 7))
        _stage_scalars(iidx_v, sub_s, b_per_w,
                       lambda v: jnp.bitwise_and(v, jnp.full((L,), TC - 1,
                                                             jnp.int32)))

        def fetch_group(e0, buf, sem):
            for j in range(G):
                off = pl.multiple_of(blk_s[e0 + j] * TC, TC)
                pltpu.async_copy(ift_hbm.at[:, pl.ds(off, TC)],
                                 buf.at[j], sem)

        def drain_group(buf, sem):
            for j in range(G):
                pltpu.make_async_copy(ift_hbm.at[:, pl.ds(0, TC)],
                                      buf.at[j], sem).wait()

        def extract_group(e0, buf):
            for j in range(G):
                e = e0 + j
                sub = jnp.full((L,), sub_s[e], jnp.int32)
                jvec = jnp.full((L,), j, jnp.int32)
                erow = jnp.full((L,), lax.shift_right_logical(e, 1),
                                jnp.int32)
                cbase = jnp.bitwise_and(e, 1) * F
                for k in range(F // L):
                    fvec = k * L + lax.iota(jnp.int32, L)
                    vals = plsc.load_gather(buf, [jvec, fvec, sub])
                    plsc.store_scatter(rows_v, [erow, cbase + fvec], vals)

        n_pairs = b_per_w // (2 * G)
        fetch_group(0, buf_a, sem_a)

        def body(p, carry):
            e = p * 2 * G
            fetch_group(e + G, buf_b, sem_b)
            drain_group(buf_a, sem_a)
            extract_group(e, buf_a)

            @pl.when(p + 1 < n_pairs)
            def _():
                fetch_group(e + 2 * G, buf_a, sem_a)

            drain_group(buf_b, sem_b)
            extract_group(e + G, buf_b)
            return carry

        lax.fori_loop(0, n_pairs, body, 0)
        pltpu.sync_copy(rows_v,
                        rows_hbm.at[pl.ds(pl.multiple_of(base // 2, 8), b_per_w // 2), :])

    @functools.partial(
        pl.kernel,
        mesh=mesh,
        out_type=jax.ShapeDtypeStruct((B,), jnp.float32),
        compiler_params=pltpu.CompilerParams(needs_layout_passes=False),
        scratch_types=[
            pltpu.VMEM((b_per_w,), jnp.int32),
            pltpu.SMEM((b_per_w,), jnp.int32),
            pltpu.VMEM((CHUNK, F), jnp.float32),
            pltpu.VMEM((b_per_w // 2, 2 * F), jnp.float32),
            pltpu.VMEM((b_per_w,), jnp.float32),
            pltpu.SemaphoreType.DMA,
            pltpu.SemaphoreType.DMA,
        ],
    )
    def dot_rows(user_hbm, uf_hbm, vrows_hbm, out_hbm,
                 uidx_v, uidx_s, urows, vrows_v, out_v, sem_u, sem_v):
        wid = lax.axis_index("s") * NC + lax.axis_index("c")
        base = wid * b_per_w
        cp_v = pltpu.async_copy(
            vrows_hbm.at[pl.ds(pl.multiple_of(base // 2, 8), b_per_w // 2), :], vrows_v, sem_v)
        pltpu.sync_copy(user_hbm.at[pl.ds(base, b_per_w)], uidx_v)
        _stage_scalars(uidx_v, uidx_s, b_per_w, lambda v: v)
        cp_v.wait()

        for c in range(b_per_w // CHUNK):

            def issue(i, carry):
                ur = uidx_s[c * CHUNK + i]
                pltpu.async_copy(uf_hbm.at[pl.ds(ur, 1)],
                                 urows.at[pl.ds(i, 1)], sem_u)
                return carry

            lax.fori_loop(0, CHUNK, issue, 0)
            pltpu.make_async_copy(uf_hbm.at[pl.ds(0, CHUNK)], urows,
                                  sem_u).wait()

            for g in range(CHUNK // L):
                rows = g * L + lax.iota(jnp.int32, L)
                il = c * CHUNK + g * L + lax.iota(jnp.int32, L)
                vrow = lax.shift_right_logical(il, 1)
                vbase = jnp.bitwise_and(il, jnp.full((L,), 1, jnp.int32)) * F
                accs = [jnp.zeros((L,), jnp.float32) for _ in range(4)]
                for f in range(F):
                    colf = jnp.full((L,), f, jnp.int32)
                    uc = plsc.load_gather(urows, [rows, colf])
                    vc = plsc.load_gather(vrows_v, [vrow, vbase + colf])
                    accs[f % 4] = accs[f % 4] + uc * vc
                acc = (accs[0] + accs[1]) + (accs[2] + accs[3])
                out_v[pl.ds(c * CHUNK + g * L, L)] = acc

        pltpu.sync_copy(out_v, out_hbm.at[pl.ds(base, b_per_w)])

    item_rows = gather_cols(item, if_t)
    return dot_rows(user, user_factors, item_rows)


# speedup vs baseline: 1.5229x; 1.0045x over previous
import functools

import jax
import jax.numpy as jnp
from jax import lax
from jax.experimental import pallas as pl
from jax.experimental.pallas import tpu as pltpu
from jax.experimental.pallas import tpu_sc as plsc

F = 64
L = 16
TC = 128
CHUNK = 256


def _stage_scalars(idx_v, dst_s, n, transform):
    lanes = lax.iota(jnp.int32, L)

    def stage(g, carry):
        vec = transform(idx_v[pl.ds(g * L, L)])
        zero = jnp.zeros((L,), jnp.int32)
        for j in range(L):
            dst_s[g * L + j] = jnp.sum(jnp.where(lanes == j, vec, zero))
        return carry

    lax.fori_loop(0, n // L, stage, 0)


@jax.jit
def kernel(user, item, user_factors, item_factors):
    B = user.shape[0]
    info = plsc.get_sparse_core_info()
    NC = info.num_cores
    NW = NC * info.num_subcores
    b_per_w = B // NW

    if_t = item_factors.T
    mesh = plsc.VectorSubcoreMesh(core_axis_name="c", subcore_axis_name="s")

    G = 4

    @functools.partial(
        pl.kernel,
        mesh=mesh,
        out_type=jax.ShapeDtypeStruct((B // 2, 2 * F), jnp.float32),
        compiler_params=pltpu.CompilerParams(needs_layout_passes=False),
        scratch_types=[
            pltpu.VMEM((b_per_w,), jnp.int32),
            pltpu.SMEM((b_per_w,), jnp.int32),
            pltpu.SMEM((b_per_w,), jnp.int32),
            pltpu.VMEM((G, F, TC), jnp.float32),
            pltpu.VMEM((G, F, TC), jnp.float32),
            pltpu.VMEM((b_per_w // 2, 2 * F), jnp.float32),
            pltpu.SemaphoreType.DMA,
            pltpu.SemaphoreType.DMA,
        ],
    )
    def gather_cols(item_hbm, ift_hbm, rows_hbm,
                    iidx_v, blk_s, sub_s, buf_a, buf_b, rows_v, sem_a, sem_b):
        wid = lax.axis_index("s") * NC + lax.axis_index("c")
        base = wid * b_per_w
        pltpu.sync_copy(item_hbm.at[pl.ds(base, b_per_w)], iidx_v)
        _stage_scalars(iidx_v, blk_s, b_per_w,
                       lambda v: lax.shift_right_logical(v, 7))
        _stage_scalars(iidx_v, sub_s, b_per_w,
                       lambda v: jnp.bitwise_and(v, jnp.full((L,), TC - 1,
                                                             jnp.int32)))

        def fetch_group(e0, buf, sem):
            for j in range(G):
                off = pl.multiple_of(blk_s[e0 + j] * TC, TC)
                pltpu.async_copy(ift_hbm.at[:, pl.ds(off, TC)],
                                 buf.at[j], sem)

        def drain_group(buf, sem):
            for j in range(G):
                pltpu.make_async_copy(ift_hbm.at[:, pl.ds(0, TC)],
                                      buf.at[j], sem).wait()

        def extract_group(e0, buf):
            for j in range(G):
                e = e0 + j
                sub = jnp.full((L,), sub_s[e], jnp.int32)
                jvec = jnp.full((L,), j, jnp.int32)
                erow = jnp.full((L,), lax.shift_right_logical(e, 1),
                                jnp.int32)
                cbase = jnp.bitwise_and(e, 1) * F
                for k in range(F // L):
                    fvec = k * L + lax.iota(jnp.int32, L)
                    vals = plsc.load_gather(buf, [jvec, fvec, sub])
                    plsc.store_scatter(rows_v, [erow, cbase + fvec], vals)

        n_pairs = b_per_w // (2 * G)
        fetch_group(0, buf_a, sem_a)

        def body(p, carry):
            e = p * 2 * G
            fetch_group(e + G, buf_b, sem_b)
            drain_group(buf_a, sem_a)
            extract_group(e, buf_a)

            @pl.when(p + 1 < n_pairs)
            def _():
                fetch_group(e + 2 * G, buf_a, sem_a)

            drain_group(buf_b, sem_b)
            extract_group(e + G, buf_b)
            return carry

        lax.fori_loop(0, n_pairs, body, 0)
        pltpu.sync_copy(rows_v,
                        rows_hbm.at[pl.ds(pl.multiple_of(base // 2, 8), b_per_w // 2), :])

    @functools.partial(
        pl.kernel,
        mesh=mesh,
        out_type=jax.ShapeDtypeStruct((B,), jnp.float32),
        compiler_params=pltpu.CompilerParams(needs_layout_passes=False),
        scratch_types=[
            pltpu.VMEM((b_per_w,), jnp.int32),
            pltpu.SMEM((b_per_w,), jnp.int32),
            pltpu.VMEM((CHUNK, F), jnp.float32),
            pltpu.VMEM((CHUNK, F), jnp.float32),
            pltpu.VMEM((b_per_w // 2, 2 * F), jnp.float32),
            pltpu.VMEM((b_per_w,), jnp.float32),
            pltpu.SemaphoreType.DMA,
            pltpu.SemaphoreType.DMA,
            pltpu.SemaphoreType.DMA,
        ],
    )
    def dot_rows(user_hbm, uf_hbm, vrows_hbm, out_hbm,
                 uidx_v, uidx_s, urows0, urows1, vrows_v, out_v,
                 sem_u0, sem_u1, sem_v):
        wid = lax.axis_index("s") * NC + lax.axis_index("c")
        base = wid * b_per_w
        cp_v = pltpu.async_copy(
            vrows_hbm.at[pl.ds(pl.multiple_of(base // 2, 8), b_per_w // 2), :], vrows_v, sem_v)
        pltpu.sync_copy(user_hbm.at[pl.ds(base, b_per_w)], uidx_v)
        _stage_scalars(uidx_v, uidx_s, b_per_w, lambda v: v)
        chunk_bufs = [(urows0, sem_u0), (urows1, sem_u1)]
        for c in range(b_per_w // CHUNK):
            buf_c, sem_c = chunk_bufs[c]

            def issue(i, carry, c=c, buf_c=buf_c, sem_c=sem_c):
                ur = uidx_s[c * CHUNK + i]
                pltpu.async_copy(uf_hbm.at[pl.ds(ur, 1)],
                                 buf_c.at[pl.ds(i, 1)], sem_c)
                return carry

            lax.fori_loop(0, CHUNK, issue, 0)
        cp_v.wait()

        for c in range(b_per_w // CHUNK):
            urows, sem_c = chunk_bufs[c]
            pltpu.make_async_copy(uf_hbm.at[pl.ds(0, CHUNK)], urows,
                                  sem_c).wait()

            for g in range(CHUNK // L):
                rows = g * L + lax.iota(jnp.int32, L)
                il = c * CHUNK + g * L + lax.iota(jnp.int32, L)
                vrow = lax.shift_right_logical(il, 1)
                vbase = jnp.bitwise_and(il, jnp.full((L,), 1, jnp.int32)) * F
                accs = [jnp.zeros((L,), jnp.float32) for _ in range(4)]
                for f in range(F):
                    colf = jnp.full((L,), f, jnp.int32)
                    uc = plsc.load_gather(urows, [rows, colf])
                    vc = plsc.load_gather(vrows_v, [vrow, vbase + colf])
                    accs[f % 4] = accs[f % 4] + uc * vc
                acc = (accs[0] + accs[1]) + (accs[2] + accs[3])
                out_v[pl.ds(c * CHUNK + g * L, L)] = acc

        pltpu.sync_copy(out_v, out_hbm.at[pl.ds(base, b_per_w)])

    item_rows = gather_cols(item, if_t)
    return dot_rows(user, user_factors, item_rows)


# speedup vs baseline: 1.5252x; 1.0015x over previous
import functools

import jax
import jax.numpy as jnp
from jax import lax
from jax.experimental import pallas as pl
from jax.experimental.pallas import tpu as pltpu
from jax.experimental.pallas import tpu_sc as plsc

F = 64
L = 16
TC = 128
CHUNK = 256


def _stage_scalars(idx_v, dst_s, n, transform):
    lanes = lax.iota(jnp.int32, L)

    def stage(g, carry):
        vec = transform(idx_v[pl.ds(g * L, L)])
        zero = jnp.zeros((L,), jnp.int32)
        for j in range(L):
            dst_s[g * L + j] = jnp.sum(jnp.where(lanes == j, vec, zero))
        return carry

    lax.fori_loop(0, n // L, stage, 0)


@jax.jit
def kernel(user, item, user_factors, item_factors):
    B = user.shape[0]
    info = plsc.get_sparse_core_info()
    NC = info.num_cores
    NW = NC * info.num_subcores
    b_per_w = B // NW

    if_t = item_factors.T
    mesh = plsc.VectorSubcoreMesh(core_axis_name="c", subcore_axis_name="s")

    G = 4

    @functools.partial(
        pl.kernel,
        mesh=mesh,
        out_type=jax.ShapeDtypeStruct((B // 2, 2 * F), jnp.float32),
        compiler_params=pltpu.CompilerParams(needs_layout_passes=False),
        scratch_types=[
            pltpu.VMEM((b_per_w,), jnp.int32),
            pltpu.SMEM((b_per_w,), jnp.int32),
            pltpu.SMEM((b_per_w,), jnp.int32),
            pltpu.VMEM((G, F, TC), jnp.float32),
            pltpu.VMEM((G, F, TC), jnp.float32),
            pltpu.VMEM((b_per_w // 2, 2 * F), jnp.float32),
            pltpu.SemaphoreType.DMA,
            pltpu.SemaphoreType.DMA,
        ],
    )
    def gather_cols(item_hbm, ift_hbm, rows_hbm,
                    iidx_v, blk_s, sub_s, buf_a, buf_b, rows_v, sem_a, sem_b):
        wid = lax.axis_index("s") * NC + lax.axis_index("c")
        base = wid * b_per_w
        pltpu.sync_copy(item_hbm.at[pl.ds(base, b_per_w)], iidx_v)
        _stage_scalars(iidx_v, blk_s, b_per_w,
                       lambda v: lax.shift_right_logical(v, 7))
        _stage_scalars(iidx_v, sub_s, b_per_w,
                       lambda v: jnp.bitwise_and(v, jnp.full((L,), TC - 1,
                                                             jnp.int32)))

        def fetch_group(e0, buf, sem):
            for j in range(G):
                off = pl.multiple_of(blk_s[e0 + j] * TC, TC)
                pltpu.async_copy(ift_hbm.at[:, pl.ds(off, TC)],
                                 buf.at[j], sem)

        def drain_group(buf, sem):
            for j in range(G):
                pltpu.make_async_copy(ift_hbm.at[:, pl.ds(0, TC)],
                                      buf.at[j], sem).wait()

        def extract_group(e0, buf):
            for j in range(G):
                e = e0 + j
                sub = jnp.full((L,), sub_s[e], jnp.int32)
                jvec = jnp.full((L,), j, jnp.int32)
                erow = jnp.full((L,), lax.shift_right_logical(e, 1),
                                jnp.int32)
                cbase = jnp.bitwise_and(e, 1) * F
                for k in range(F // L):
                    fvec = k * L + lax.iota(jnp.int32, L)
                    vals = plsc.load_gather(buf, [jvec, fvec, sub])
                    plsc.store_scatter(rows_v, [erow, cbase + fvec], vals)

        n_pairs = b_per_w // (2 * G)
        fetch_group(0, buf_a, sem_a)

        def body(p, carry):
            e = p * 2 * G
            fetch_group(e + G, buf_b, sem_b)
            drain_group(buf_a, sem_a)
            extract_group(e, buf_a)

            @pl.when(p + 1 < n_pairs)
            def _():
                fetch_group(e + 2 * G, buf_a, sem_a)

            drain_group(buf_b, sem_b)
            extract_group(e + G, buf_b)
            return carry

        lax.fori_loop(0, n_pairs, body, 0)
        pltpu.sync_copy(rows_v,
                        rows_hbm.at[pl.ds(pl.multiple_of(base // 2, 8), b_per_w // 2), :])

    @functools.partial(
        pl.kernel,
        mesh=mesh,
        out_type=jax.ShapeDtypeStruct((B,), jnp.float32),
        compiler_params=pltpu.CompilerParams(needs_layout_passes=False),
        scratch_types=[
            pltpu.VMEM((b_per_w,), jnp.int32),
            pltpu.SMEM((b_per_w,), jnp.int32),
            pltpu.VMEM((CHUNK, F), jnp.float32),
            pltpu.VMEM((CHUNK, F), jnp.float32),
            pltpu.VMEM((b_per_w // 2, 2 * F), jnp.float32),
            pltpu.VMEM((b_per_w,), jnp.float32),
            pltpu.SemaphoreType.DMA,
            pltpu.SemaphoreType.DMA,
            pltpu.SemaphoreType.DMA,
        ],
    )
    def dot_rows(user_hbm, uf_hbm, vrows_hbm, out_hbm,
                 uidx_v, uidx_s, urows0, urows1, vrows_v, out_v,
                 sem_u0, sem_u1, sem_v):
        wid = lax.axis_index("s") * NC + lax.axis_index("c")
        base = wid * b_per_w
        cp_v = pltpu.async_copy(
            vrows_hbm.at[pl.ds(pl.multiple_of(base // 2, 8), b_per_w // 2), :], vrows_v, sem_v)
        pltpu.sync_copy(user_hbm.at[pl.ds(base, b_per_w)], uidx_v)
        _stage_scalars(uidx_v, uidx_s, b_per_w, lambda v: v)
        chunk_bufs = [(urows0, sem_u0), (urows1, sem_u1)]
        for c in range(b_per_w // CHUNK):
            buf_c, sem_c = chunk_bufs[c]

            def issue(q, carry, c=c, buf_c=buf_c, sem_c=sem_c):
                for j in range(4):
                    i = q * 4 + j
                    ur = uidx_s[c * CHUNK + i]
                    pltpu.async_copy(uf_hbm.at[pl.ds(ur, 1)],
                                     buf_c.at[pl.ds(i, 1)], sem_c)
                return carry

            lax.fori_loop(0, CHUNK // 4, issue, 0)
        cp_v.wait()

        for c in range(b_per_w // CHUNK):
            urows, sem_c = chunk_bufs[c]
            pltpu.make_async_copy(uf_hbm.at[pl.ds(0, CHUNK)], urows,
                                  sem_c).wait()

            for g in range(CHUNK // L):
                rows = g * L + lax.iota(jnp.int32, L)
                il = c * CHUNK + g * L + lax.iota(jnp.int32, L)
                vrow = lax.shift_right_logical(il, 1)
                vbase = jnp.bitwise_and(il, jnp.full((L,), 1, jnp.int32)) * F
                accs = [jnp.zeros((L,), jnp.float32) for _ in range(4)]
                for f in range(F):
                    colf = jnp.full((L,), f, jnp.int32)
                    uc = plsc.load_gather(urows, [rows, colf])
                    vc = plsc.load_gather(vrows_v, [vrow, vbase + colf])
                    accs[f % 4] = accs[f % 4] + uc * vc
                acc = (accs[0] + accs[1]) + (accs[2] + accs[3])
                out_v[pl.ds(c * CHUNK + g * L, L)] = acc

        pltpu.sync_copy(out_v, out_hbm.at[pl.ds(base, b_per_w)])

    item_rows = gather_cols(item, if_t)
    return dot_rows(user, user_factors, item_rows)
